# baseline (speedup 1.0000x reference)
import jax
import jax.numpy as jnp
from jax import lax
from jax.experimental import pallas as pl

VOCAB = 1000
HIDDEN = 128
B = 1024
L = 50
TB = 8
NT = B // TB


def _kernel(ids_ref, emb_ref, pw_ref, pb_ref, rw_ref, rb_ref,
            out_ref, pred_ref):
    ids = ids_ref[...]
    v = lax.broadcasted_iota(jnp.int32, (TB, L, VOCAB), 2)
    oh = (ids == v).astype(jnp.float32)
    hid = lax.dot_general(
        oh, emb_ref[...], (((2,), (0,)), ((), ())),
        preferred_element_type=jnp.float32)
    out_ref[...] = lax.dot_general(
        hid, pw_ref[...], (((2,), (0,)), ((), ())),
        preferred_element_type=jnp.float32) + pb_ref[...]
    pooled = jnp.mean(hid, axis=1)
    pred_ref[...] = lax.dot_general(
        pooled, rw_ref[...], (((1,), (0,)), ((), ())),
        preferred_element_type=jnp.float32) + rb_ref[...]


def kernel(input_ids, embed_table, proj_w, proj_b, reg_w, reg_b):
    ids3 = input_ids.astype(jnp.int32).reshape(B, L, 1)
    logits, pred = pl.pallas_call(
        _kernel,
        grid=(NT,),
        in_specs=[
            pl.BlockSpec((TB, L, 1), lambda i: (i, 0, 0)),
            pl.BlockSpec((VOCAB, HIDDEN), lambda i: (0, 0)),
            pl.BlockSpec((HIDDEN, VOCAB), lambda i: (0, 0)),
            pl.BlockSpec((1, 1, VOCAB), lambda i: (0, 0, 0)),
            pl.BlockSpec((HIDDEN, 1), lambda i: (0, 0)),
            pl.BlockSpec((1, 1), lambda i: (0, 0)),
        ],
        out_specs=(
            pl.BlockSpec((TB, L, VOCAB), lambda i: (i, 0, 0)),
            pl.BlockSpec((TB, 1), lambda i: (i, 0)),
        ),
        out_shape=(
            jax.ShapeDtypeStruct((B, L, VOCAB), jnp.float32),
            jax.ShapeDtypeStruct((B, 1), jnp.float32),
        ),
    )(ids3, embed_table, proj_w, proj_b.reshape(1, 1, VOCAB),
      reg_w, reg_b.reshape(1, 1))
    return logits, pred


# speedup vs baseline: 1.1000x; 1.1000x over previous
import jax
import jax.numpy as jnp
from jax import lax
from jax.experimental import pallas as pl

VOCAB = 1000
HIDDEN = 128
B = 1024
L = 50
TB = 32
NT = B // TB


def _kernel(ids_ref, emb_ref, pw_ref, pb_ref, rw_ref, rb_ref,
            out_ref, pred_ref):
    ids = ids_ref[...]
    v = lax.broadcasted_iota(jnp.int32, (TB, L, VOCAB), 2)
    oh = (ids == v).astype(jnp.float32)
    hid = lax.dot_general(
        oh, emb_ref[...], (((2,), (0,)), ((), ())),
        preferred_element_type=jnp.float32)
    out_ref[...] = lax.dot_general(
        hid, pw_ref[...], (((2,), (0,)), ((), ())),
        preferred_element_type=jnp.float32) + pb_ref[...]
    pooled = jnp.mean(hid, axis=1)
    pred_ref[...] = lax.dot_general(
        pooled, rw_ref[...], (((1,), (0,)), ((), ())),
        preferred_element_type=jnp.float32) + rb_ref[...]


def kernel(input_ids, embed_table, proj_w, proj_b, reg_w, reg_b):
    ids3 = input_ids.astype(jnp.int32).reshape(B, L, 1)
    logits, pred = pl.pallas_call(
        _kernel,
        grid=(NT,),
        in_specs=[
            pl.BlockSpec((TB, L, 1), lambda i: (i, 0, 0)),
            pl.BlockSpec((VOCAB, HIDDEN), lambda i: (0, 0)),
            pl.BlockSpec((HIDDEN, VOCAB), lambda i: (0, 0)),
            pl.BlockSpec((1, 1, VOCAB), lambda i: (0, 0, 0)),
            pl.BlockSpec((HIDDEN, 1), lambda i: (0, 0)),
            pl.BlockSpec((1, 1), lambda i: (0, 0)),
        ],
        out_specs=(
            pl.BlockSpec((TB, L, VOCAB), lambda i: (i, 0, 0)),
            pl.BlockSpec((TB, 1), lambda i: (i, 0)),
        ),
        out_shape=(
            jax.ShapeDtypeStruct((B, L, VOCAB), jnp.float32),
            jax.ShapeDtypeStruct((B, 1), jnp.float32),
        ),
    )(ids3, embed_table, proj_w, proj_b.reshape(1, 1, VOCAB),
      reg_w, reg_b.reshape(1, 1))
    return logits, pred


# speedup vs baseline: 1.1037x; 1.0034x over previous
import jax
import jax.numpy as jnp
from jax import lax
from jax.experimental import pallas as pl

VOCAB = 1000
HIDDEN = 128
B = 1024
L = 50
TB = 64
NT = B // TB


def _kernel(ids_ref, emb_ref, pw_ref, pb_ref, rw_ref, rb_ref,
            out_ref, pred_ref):
    ids = ids_ref[...]
    v = lax.broadcasted_iota(jnp.int32, (TB, L, VOCAB), 2)
    oh = (ids == v).astype(jnp.float32)
    hid = lax.dot_general(
        oh, emb_ref[...], (((2,), (0,)), ((), ())),
        preferred_element_type=jnp.float32)
    out_ref[...] = lax.dot_general(
        hid, pw_ref[...], (((2,), (0,)), ((), ())),
        preferred_element_type=jnp.float32) + pb_ref[...]
    pooled = jnp.mean(hid, axis=1)
    pred_ref[...] = lax.dot_general(
        pooled, rw_ref[...], (((1,), (0,)), ((), ())),
        preferred_element_type=jnp.float32) + rb_ref[...]


def kernel(input_ids, embed_table, proj_w, proj_b, reg_w, reg_b):
    ids3 = input_ids.astype(jnp.int32).reshape(B, L, 1)
    logits, pred = pl.pallas_call(
        _kernel,
        grid=(NT,),
        in_specs=[
            pl.BlockSpec((TB, L, 1), lambda i: (i, 0, 0)),
            pl.BlockSpec((VOCAB, HIDDEN), lambda i: (0, 0)),
            pl.BlockSpec((HIDDEN, VOCAB), lambda i: (0, 0)),
            pl.BlockSpec((1, 1, VOCAB), lambda i: (0, 0, 0)),
            pl.BlockSpec((HIDDEN, 1), lambda i: (0, 0)),
            pl.BlockSpec((1, 1), lambda i: (0, 0)),
        ],
        out_specs=(
            pl.BlockSpec((TB, L, VOCAB), lambda i: (i, 0, 0)),
            pl.BlockSpec((TB, 1), lambda i: (i, 0)),
        ),
        out_shape=(
            jax.ShapeDtypeStruct((B, L, VOCAB), jnp.float32),
            jax.ShapeDtypeStruct((B, 1), jnp.float32),
        ),
    )(ids3, embed_table, proj_w, proj_b.reshape(1, 1, VOCAB),
      reg_w, reg_b.reshape(1, 1))
    return logits, pred
